# baseline (speedup 1.0000x reference)
import functools

import jax
import jax.numpy as jnp
from jax.experimental import pallas as pl
from jax.experimental.pallas import tpu as pltpu

_N = 10000
_BM = 200
_NBUF = 4


def _gcn_body(x_ref, w_ref, adj_hbm, out_ref, support_ref, bufs_ref, sem_ref):
    nblk = _N // _BM

    def _start(i):
        slot = i % _NBUF
        pltpu.make_async_copy(
            adj_hbm.at[pl.ds(i * _BM, _BM), :],
            bufs_ref.at[slot],
            sem_ref.at[slot],
        ).start()

    for i in range(_NBUF):
        _start(i)

    support_ref[...] = jnp.dot(
        x_ref[...], w_ref[...], preferred_element_type=jnp.float32
    )

    for i in range(nblk):
        slot = i % _NBUF
        pltpu.make_async_copy(
            adj_hbm.at[pl.ds(i * _BM, _BM), :],
            bufs_ref.at[slot],
            sem_ref.at[slot],
        ).wait()
        out_ref[i * _BM:(i + 1) * _BM, :] = jnp.dot(
            bufs_ref[slot].astype(jnp.bfloat16),
            support_ref[...].astype(jnp.bfloat16),
            preferred_element_type=jnp.float32,
        )
        j = i + _NBUF
        if j < nblk:
            _start(j)


@functools.partial(jax.jit, static_argnames=())
def kernel(input, adj, W):
    n, in_f = input.shape
    out_f = W.shape[1]
    return pl.pallas_call(
        _gcn_body,
        in_specs=[
            pl.BlockSpec(memory_space=pltpu.VMEM),
            pl.BlockSpec(memory_space=pltpu.VMEM),
            pl.BlockSpec(memory_space=pl.ANY),
        ],
        out_specs=pl.BlockSpec(memory_space=pltpu.VMEM),
        out_shape=jax.ShapeDtypeStruct((n, out_f), jnp.float32),
        scratch_shapes=[
            pltpu.VMEM((n, out_f), jnp.float32),
            pltpu.VMEM((_NBUF, _BM, n), jnp.float32),
            pltpu.SemaphoreType.DMA((_NBUF,)),
        ],
    )(input, W, adj)
